# baseline (speedup 1.0000x reference)
import jax
import jax.numpy as jnp
from jax import lax
from jax.experimental import pallas as pl
from jax.experimental.pallas import tpu as pltpu

from kernel_tc_test import tc_pool_from_idx

_D = 128
_B = 4096
_HIST = 50


def _mlp_kernel(x_ref, w1_ref, b1_ref, w2_ref, b2_ref, w3_ref, b3_ref, o_ref):
    x = x_ref[...]
    sq = jnp.sum(x * x, axis=1, keepdims=True)
    xn = x * lax.rsqrt(jnp.maximum(sq, 1e-4))
    h1 = jnp.maximum(
        jnp.dot(xn, w1_ref[...], preferred_element_type=jnp.float32) + b1_ref[...], 0.0)
    h2 = jnp.maximum(
        jnp.dot(h1, w2_ref[...], preferred_element_type=jnp.float32) + b2_ref[...], 0.0)
    o_ref[...] = (
        jnp.dot(h2, w3_ref[...], preferred_element_type=jnp.float32) + b3_ref[...])


def kernel(inputs, embeddings, W1, b1, W2, b2, W3, b3):
    pooled = tc_pool_from_idx(inputs, embeddings)

    blk = 512
    out = pl.pallas_call(
        _mlp_kernel,
        grid=(_B // blk,),
        in_specs=[
            pl.BlockSpec((blk, _D), lambda i: (i, 0)),
            pl.BlockSpec((_D, 2 * _D), lambda i: (0, 0)),
            pl.BlockSpec((1, 2 * _D), lambda i: (0, 0)),
            pl.BlockSpec((2 * _D, 4 * _D), lambda i: (0, 0)),
            pl.BlockSpec((1, 4 * _D), lambda i: (0, 0)),
            pl.BlockSpec((4 * _D, _D), lambda i: (0, 0)),
            pl.BlockSpec((1, _D), lambda i: (0, 0)),
        ],
        out_specs=pl.BlockSpec((blk, _D), lambda i: (i, 0)),
        out_shape=jax.ShapeDtypeStruct((_B, _D), jnp.float32),
    )(pooled, W1, b1.reshape(1, -1), W2, b2.reshape(1, -1), W3,
      b3.reshape(1, -1))
    return out


# speedup vs baseline: 1367.3444x; 1367.3444x over previous
import jax
import jax.numpy as jnp
from jax import lax
from jax.experimental import pallas as pl
from jax.experimental.pallas import tpu as pltpu
from jax.experimental.pallas import tpu_sc as plsc

_D = 128
_B = 4096
_HIST = 50
_HP = 56
_NC, _NS = 2, 16
_NW = _NC * _NS
_BPW = _B // _NW
_SPW = _BPW * _HP
_RPC = 2
_IPC = _RPC * _HIST
_NCH = _BPW // _RPC
_L = 16


def _sc_pool(idx_hbm, scale_hbm, emb_hbm, out_hbm, idx_v, scale_v,
             g0, g1, g2, g3, e0_v, out_v, sem0, sem1, sem2, sem3):
    cid = lax.axis_index("c")
    sid = lax.axis_index("s")
    wid = sid * _NC + cid
    pltpu.sync_copy(idx_hbm.at[pl.ds(wid * _NCH, _NCH), :], idx_v)
    pltpu.sync_copy(scale_hbm.at[pl.ds(wid * _SPW, _SPW)],
                    scale_v.at[pl.ds(0, _SPW)])
    pltpu.sync_copy(emb_hbm.at[pl.ds(0, 1), :], e0_v)

    one = jnp.ones((_L,), jnp.float32)
    zero = jnp.zeros((_L,), jnp.float32)
    scale_v[pl.ds(_SPW, _L)] = zero

    def start_gather(k, gbuf, sem):
        pltpu.async_copy(emb_hbm.at[idx_v.at[k]], gbuf, sem)

    def wait_gather(k, gbuf, sem):
        pltpu.make_async_copy(emb_hbm.at[idx_v.at[k]], gbuf, sem).wait()

    tail_mask = jnp.where(lax.iota(jnp.int32, _L) < 8, one, zero)

    def accum(k, gbuf):
        for sb in range(_RPC):
            fbase = k * (_RPC * _HP) + sb * _HP
            svecs = [scale_v[pl.ds(fbase + _L * j, _L)] for j in range(4)]
            ssum = svecs[0] + svecs[1] + svecs[2] + svecs[3] * tail_mask
            for shift in (8, 4, 2, 1):
                perm = lax.iota(jnp.int32, _L) ^ shift
                ssum = ssum + lax.gather(
                    ssum, perm[:, None],
                    lax.GatherDimensionNumbers(
                        offset_dims=(), collapsed_slice_dims=(0,),
                        start_index_map=(0,)),
                    (1,), mode=lax.GatherScatterMode.PROMISE_IN_BOUNDS)
            n0 = jnp.float32(_HIST) - ssum
            accs = [jnp.zeros((_L,), jnp.float32) for _ in range(_D // _L)]
            for r in range(_HIST):
                for c in range(_D // _L):
                    accs[c] = accs[c] + gbuf[sb * _HIST + r, pl.ds(c * _L, _L)]
            row = k * _RPC + sb
            for c in range(_D // _L):
                out_v[row, pl.ds(c * _L, _L)] = (
                    accs[c] - n0 * e0_v[0, pl.ds(c * _L, _L)])

    gs = [g0, g1, g2, g3]
    sems = [sem0, sem1, sem2, sem3]
    nb = len(gs)
    for k in range(nb - 1):
        start_gather(k, gs[k], sems[k])

    def chunk_body(kb, carry):
        for par in range(nb):
            k = kb * nb + par
            nxt = (par + nb - 1) % nb

            @pl.when(k + nb - 1 < _NCH)
            def _():
                start_gather(k + nb - 1, gs[nxt], sems[nxt])

            wait_gather(k, gs[par], sems[par])
            accum(k, gs[par])
        return carry

    lax.fori_loop(0, _NCH // nb, chunk_body, 0)
    pltpu.sync_copy(out_v, out_hbm.at[pl.ds(wid * _BPW, _BPW), :])


def _mlp_kernel(x_ref, w1_ref, b1_ref, w2_ref, b2_ref, w3_ref, b3_ref, o_ref):
    x = x_ref[...]
    sq = jnp.sum(x * x, axis=1, keepdims=True)
    xn = x * lax.rsqrt(jnp.maximum(sq, 1e-4))
    h1 = jnp.maximum(
        jnp.dot(xn, w1_ref[...], preferred_element_type=jnp.float32) + b1_ref[...], 0.0)
    h2 = jnp.maximum(
        jnp.dot(h1, w2_ref[...], preferred_element_type=jnp.float32) + b2_ref[...], 0.0)
    o_ref[...] = (
        jnp.dot(h2, w3_ref[...], preferred_element_type=jnp.float32) + b3_ref[...])


def kernel(inputs, embeddings, W1, b1, W2, b2, W3, b3):
    idx = inputs.astype(jnp.int32)
    idx_adj = (jnp.maximum(idx, 1) - 1).reshape(_B // _RPC, _IPC)
    scale = jnp.pad((idx > 0).astype(jnp.float32),
                    ((0, 0), (0, _HP - _HIST))).reshape(-1)

    mesh = plsc.VectorSubcoreMesh(core_axis_name="c", subcore_axis_name="s")
    pooled = pl.kernel(
        _sc_pool,
        out_type=jax.ShapeDtypeStruct((_B, _D), jnp.float32),
        mesh=mesh,
        scratch_types=[
            pltpu.VMEM((_NCH, _IPC), jnp.int32),
            pltpu.VMEM((_SPW + _L,), jnp.float32),
            pltpu.VMEM((_IPC, _D), jnp.float32),
            pltpu.VMEM((_IPC, _D), jnp.float32),
            pltpu.VMEM((_IPC, _D), jnp.float32),
            pltpu.VMEM((_IPC, _D), jnp.float32),
            pltpu.VMEM((1, _D), jnp.float32),
            pltpu.VMEM((_BPW, _D), jnp.float32),
            pltpu.SemaphoreType.DMA,
            pltpu.SemaphoreType.DMA,
            pltpu.SemaphoreType.DMA,
            pltpu.SemaphoreType.DMA,
        ],
    )(idx_adj, scale, embeddings)

    blk = 512
    out = pl.pallas_call(
        _mlp_kernel,
        grid=(_B // blk,),
        in_specs=[
            pl.BlockSpec((blk, _D), lambda i: (i, 0)),
            pl.BlockSpec((_D, 2 * _D), lambda i: (0, 0)),
            pl.BlockSpec((1, 2 * _D), lambda i: (0, 0)),
            pl.BlockSpec((2 * _D, 4 * _D), lambda i: (0, 0)),
            pl.BlockSpec((1, 4 * _D), lambda i: (0, 0)),
            pl.BlockSpec((4 * _D, _D), lambda i: (0, 0)),
            pl.BlockSpec((1, _D), lambda i: (0, 0)),
        ],
        out_specs=pl.BlockSpec((blk, _D), lambda i: (i, 0)),
        out_shape=jax.ShapeDtypeStruct((_B, _D), jnp.float32),
    )(pooled, W1, b1.reshape(1, -1), W2, b2.reshape(1, -1), W3,
      b3.reshape(1, -1))
    return out
